# baseline (speedup 1.0000x reference)
import functools

import jax
import jax.numpy as jnp
from jax.experimental import pallas as pl
from jax.experimental.pallas import tpu as pltpu


def _heads_kernel(oc, od, hb, w_ref, b_ref, x_ref, oc_ref, or_ref, od_ref):
    dn = (((1,), (0,)), ((), ()))
    for h in range(hb):
        x = x_ref[0, :, h, :]
        y = jax.lax.dot_general(
            w_ref[...], x, dn, preferred_element_type=jnp.float32)
        y = y + b_ref[...]
        oc_ref[0, :, h, :] = y[:oc]
        or_ref[0, :, h, :] = y[oc:-od]
        od_ref[0, :, h, :] = y[-od:]


def kernel(x, W_cls, b_cls, W_reg, b_reg, W_dir, b_dir):
    B, C, H, W = x.shape
    Oc, Or, Od = W_cls.shape[0], W_reg.shape[0], W_dir.shape[0]
    Ot = Oc + Or + Od
    HB = 8
    assert H % HB == 0

    w_all = jnp.concatenate([W_cls, W_reg, W_dir], axis=0)
    b_all = jnp.concatenate([b_cls, b_reg, b_dir], axis=0)[:, None]

    body = functools.partial(_heads_kernel, Oc, Od, HB)
    full = lambda shape: pl.BlockSpec(shape, lambda b, h: (0, 0))
    outs = pl.pallas_call(
        body,
        grid=(B, H // HB),
        in_specs=[
            full((Ot, C)), full((Ot, 1)),
            pl.BlockSpec((1, C, HB, W), lambda b, h: (b, 0, h, 0)),
        ],
        out_specs=[
            pl.BlockSpec((1, Oc, HB, W), lambda b, h: (b, 0, h, 0)),
            pl.BlockSpec((1, Or, HB, W), lambda b, h: (b, 0, h, 0)),
            pl.BlockSpec((1, Od, HB, W), lambda b, h: (b, 0, h, 0)),
        ],
        out_shape=[
            jax.ShapeDtypeStruct((B, Oc, H, W), jnp.float32),
            jax.ShapeDtypeStruct((B, Or, H, W), jnp.float32),
            jax.ShapeDtypeStruct((B, Od, H, W), jnp.float32),
        ],
        compiler_params=pltpu.CompilerParams(
            dimension_semantics=("parallel", "parallel")),
    )(w_all, b_all, x)
    return tuple(outs)
